# Initial kernel scaffold; baseline (speedup 1.0000x reference)
import jax
import jax.numpy as jnp
from jax.experimental import pallas as pl


def kernel(index, targets, token_embedding_table):
    raise NotImplementedError("write your pallas kernel here")


# speedup vs baseline: 1.3510x; 1.3510x over previous
import functools

import jax
import jax.numpy as jnp
from jax import lax
from jax.experimental import pallas as pl
from jax.experimental.pallas import tpu as pltpu
from jax.experimental.pallas import tpu_sc as plsc

VOCAB = 1000
T_DIM = 200
B_DIM = 4096
NTOK = T_DIM * B_DIM

NC, NS, L = 2, 16, 16
NW = NC * NS
NP = NTOK // NW
K = 128
G = NP // K


def _lse_body(t_ref, o_ref):
    x = t_ref[...]
    m = jnp.max(x, axis=1, keepdims=True)
    o_ref[...] = m + jnp.log(jnp.sum(jnp.exp(x - m), axis=1, keepdims=True))


_MESH = plsc.VectorSubcoreMesh(
    core_axis_name="c", subcore_axis_name="s", num_cores=NC, num_subcores=NS
)


@functools.partial(
    pl.kernel,
    out_type=[
        jax.ShapeDtypeStruct((NTOK, VOCAB), jnp.float32),
        jax.ShapeDtypeStruct((NW, L), jnp.float32),
    ],
    mesh=_MESH,
    compiler_params=pltpu.CompilerParams(
        needs_layout_passes=False, use_tc_tiling_on_sc=False
    ),
    scratch_types=[
        pltpu.VMEM((K,), jnp.int32),
        pltpu.VMEM((K,), jnp.int32),
        pltpu.VMEM((VOCAB,), jnp.float32),
        pltpu.VMEM((K, VOCAB), jnp.float32),
        pltpu.VMEM((L,), jnp.float32),
        pltpu.SemaphoreType.DMA,
    ],
)
def _sc_gather_loss(idx_hbm, tgt_hbm, table_hbm, lse_hbm, out_hbm, part_hbm,
                    idx_v, tgt_v, lse_v, rows_v, acc_v, gsem):
    wid = lax.axis_index("s") * NC + lax.axis_index("c")
    wbase = wid * NP
    pltpu.sync_copy(lse_hbm, lse_v)

    def body(g, acc):
        base = wbase + g * K
        pltpu.sync_copy(idx_hbm.at[pl.ds(base, K)], idx_v)
        pltpu.sync_copy(tgt_hbm.at[pl.ds(base, K)], tgt_v)
        pltpu.async_copy(table_hbm.at[idx_v], rows_v, gsem).wait()
        for j in range(K // L):
            iv = idx_v[pl.ds(j * L, L)]
            tv = tgt_v[pl.ds(j * L, L)]
            lsev = plsc.load_gather(lse_v, [iv])
            kvec = lax.iota(jnp.int32, L) + (j * L)
            tabv = plsc.load_gather(rows_v, [kvec, tv])
            acc = acc + (lsev - tabv)
        pltpu.sync_copy(rows_v, out_hbm.at[pl.ds(base, K)])
        return acc

    acc = lax.fori_loop(0, G, body, jnp.zeros((L,), jnp.float32))
    acc_v[...] = acc
    pltpu.sync_copy(acc_v, part_hbm.at[wid])


def kernel(index, targets, token_embedding_table):
    idx_flat = jnp.transpose(index, (1, 0)).reshape(-1).astype(jnp.int32)
    tgt_flat = targets.reshape(-1).astype(jnp.int32)
    table = token_embedding_table

    lse = pl.pallas_call(
        _lse_body,
        out_shape=jax.ShapeDtypeStruct((VOCAB, 1), jnp.float32),
    )(table).reshape(VOCAB)

    out_flat, parts = _sc_gather_loss(idx_flat, tgt_flat, table, lse)
    logits = out_flat.reshape(B_DIM, T_DIM, VOCAB)
    loss = jnp.sum(parts) / jnp.float32(NTOK * T_DIM)
    return logits, loss


# speedup vs baseline: 1.3913x; 1.0298x over previous
import functools

import jax
import jax.numpy as jnp
from jax import lax
from jax.experimental import pallas as pl
from jax.experimental.pallas import tpu as pltpu
from jax.experimental.pallas import tpu_sc as plsc

VOCAB = 1000
T_DIM = 200
B_DIM = 4096
NTOK = T_DIM * B_DIM

NC, NS, L = 2, 16, 16
NW = NC * NS
NP = NTOK // NW
K = 64
G = NP // K


def _lse_body(t_ref, o_ref):
    x = t_ref[...]
    m = jnp.max(x, axis=1, keepdims=True)
    o_ref[...] = m + jnp.log(jnp.sum(jnp.exp(x - m), axis=1, keepdims=True))


_MESH = plsc.VectorSubcoreMesh(
    core_axis_name="c", subcore_axis_name="s", num_cores=NC, num_subcores=NS
)


@functools.partial(
    pl.kernel,
    out_type=[
        jax.ShapeDtypeStruct((NTOK, VOCAB), jnp.float32),
        jax.ShapeDtypeStruct((NW, L), jnp.float32),
    ],
    mesh=_MESH,
    compiler_params=pltpu.CompilerParams(
        needs_layout_passes=False, use_tc_tiling_on_sc=False
    ),
    scratch_types=[
        pltpu.VMEM((K,), jnp.int32),
        pltpu.VMEM((K,), jnp.int32),
        pltpu.VMEM((K,), jnp.int32),
        pltpu.VMEM((K,), jnp.int32),
        pltpu.VMEM((VOCAB,), jnp.float32),
        pltpu.VMEM((K, VOCAB), jnp.float32),
        pltpu.VMEM((K, VOCAB), jnp.float32),
        pltpu.VMEM((L,), jnp.float32),
        pltpu.SemaphoreType.DMA,
        pltpu.SemaphoreType.DMA,
        pltpu.SemaphoreType.DMA,
        pltpu.SemaphoreType.DMA,
    ],
)
def _sc_gather_loss(idx_hbm, tgt_hbm, table_hbm, lse_hbm, out_hbm, part_hbm,
                    idx_v0, idx_v1, tgt_v0, tgt_v1, lse_v, rows_v0, rows_v1,
                    acc_v, gsem0, gsem1, ssem0, ssem1):
    wid = lax.axis_index("s") * NC + lax.axis_index("c")
    wbase = wid * NP
    idxs, tgts = (idx_v0, idx_v1), (tgt_v0, tgt_v1)
    rows, gsems, ssems = (rows_v0, rows_v1), (gsem0, gsem1), (ssem0, ssem1)
    pltpu.sync_copy(lse_hbm, lse_v)

    def fetch(slot, base):
        pltpu.sync_copy(idx_hbm.at[pl.ds(base, K)], idxs[slot])
        pltpu.sync_copy(tgt_hbm.at[pl.ds(base, K)], tgts[slot])
        pltpu.async_copy(table_hbm.at[idxs[slot]], rows[slot], gsems[slot])

    fetch(0, wbase)

    def pair_body(gg, acc):
        for s in (0, 1):
            g = gg * 2 + s
            s2 = 1 - s

            @pl.when(g > 0)
            def _():
                pltpu.make_async_copy(
                    rows[s2], out_hbm.at[pl.ds(wbase, K)], ssems[s2]
                ).wait()

            @pl.when(g < G - 1)
            def _():
                fetch(s2, wbase + (g + 1) * K)

            pltpu.make_async_copy(
                table_hbm.at[idxs[s]], rows[s], gsems[s]
            ).wait()
            for j in range(K // L):
                iv = idxs[s][pl.ds(j * L, L)]
                tv = tgts[s][pl.ds(j * L, L)]
                lsev = plsc.load_gather(lse_v, [iv])
                kvec = lax.iota(jnp.int32, L) + (j * L)
                tabv = plsc.load_gather(rows[s], [kvec, tv])
                acc = acc + (lsev - tabv)
            pltpu.async_copy(
                rows[s], out_hbm.at[pl.ds(wbase + g * K, K)], ssems[s]
            )
        return acc

    acc = lax.fori_loop(0, G // 2, pair_body, jnp.zeros((L,), jnp.float32))
    pltpu.make_async_copy(rows[1], out_hbm.at[pl.ds(wbase, K)], ssems[1]).wait()
    acc_v[...] = acc
    pltpu.sync_copy(acc_v, part_hbm.at[wid])


def kernel(index, targets, token_embedding_table):
    idx_flat = jnp.transpose(index, (1, 0)).reshape(-1).astype(jnp.int32)
    tgt_flat = targets.reshape(-1).astype(jnp.int32)
    table = token_embedding_table

    lse = pl.pallas_call(
        _lse_body,
        out_shape=jax.ShapeDtypeStruct((VOCAB, 1), jnp.float32),
    )(table).reshape(VOCAB)

    out_flat, parts = _sc_gather_loss(idx_flat, tgt_flat, table, lse)
    logits = out_flat.reshape(B_DIM, T_DIM, VOCAB)
    loss = jnp.sum(parts) / jnp.float32(NTOK * T_DIM)
    return logits, loss
